# baseline (speedup 1.0000x reference)
import functools

import jax
import jax.numpy as jnp
from jax import lax
from jax.experimental import pallas as pl
from jax.experimental.pallas import tpu as pltpu
from jax.experimental.pallas import tpu_sc as plsc

_NCORES = 2
_NSUB = 16
_NW = _NCORES * _NSUB
_L = 16
_BIG = 2 ** 30


def _recip_q_body(noise_ref, r_ref):
    q = -jnp.log(jnp.clip(noise_ref[...], 1e-10, 1.0))
    r_ref[...] = 1.0 / q


def _make_recip_q(b, v):
    n = b * v
    assert n % 128 == 0
    rows = n // 128
    grid = 1
    for g in (10, 8, 5, 4, 2):
        if rows % g == 0 and (rows // g) % 8 == 0:
            grid = g
            break
    blk = rows // grid
    call = pl.pallas_call(
        _recip_q_body,
        out_shape=jax.ShapeDtypeStruct((rows, 128), jnp.float32),
        grid=(grid,),
        in_specs=[pl.BlockSpec((blk, 128), lambda i: (i, 0))],
        out_specs=pl.BlockSpec((blk, 128), lambda i: (i, 0)),
    )

    def run(noise):
        return call(noise.reshape(rows, 128)).reshape(b, v)

    return run


def _pick_chunk(v):
    best = _L
    c = _L
    while c <= 4000:
        if v % c == 0:
            best = c
        c += _L
    return best


def _make_sc(b, s, v):
    s1 = s + 1
    c_sz = _pick_chunk(v)
    nch = v // c_sz
    nvec = c_sz // _L
    rows = b // _NW
    nstr = s1 + s
    mesh = plsc.VectorSubcoreMesh(core_axis_name="c", subcore_axis_name="s",
                                  num_cores=_NCORES, num_subcores=_NSUB)

    @functools.partial(
        pl.kernel,
        out_type=jax.ShapeDtypeStruct((b, 16), jnp.int32),
        mesh=mesh,
        compiler_params=pltpu.CompilerParams(use_tc_tiling_on_sc=False,
                                             needs_layout_passes=False),
        scratch_types=[
            pltpu.VMEM((s1, c_sz), jnp.float32),
            pltpu.VMEM((c_sz,), jnp.float32),
            pltpu.VMEM((nstr * nch * _L,), jnp.float32),
            pltpu.VMEM((16,), jnp.int32),
            pltpu.VMEM((16,), jnp.float32),
            pltpu.VMEM((16,), jnp.int32),
        ],
    )
    def sc_kernel(p_hbm, r_hbm, mi_hbm, mf_hbm, out_hbm,
                  pbuf, rbuf, cmax, mi, mf, outv):
        wid = lax.axis_index("s") * _NCORES + lax.axis_index("c")
        iota = lax.iota(jnp.int32, _L)
        zero16 = jnp.zeros((_L,), jnp.float32)
        bigvec = jnp.full((_L,), _BIG, jnp.int32)

        for rr in range(rows):
            brow = wid * rows + rr
            pltpu.sync_copy(mi_hbm.at[brow], mi)
            pltpu.sync_copy(mf_hbm.at[brow], mf)
            miv = mi[...]
            mfv = mf[...]
            drafts = [miv[ss] for ss in range(s)]

            def chunk_body(ci, carry):
                gathv = carry[0]
                gaccs = carry[1:]
                off = ci * c_sz
                pltpu.sync_copy(p_hbm.at[brow, :, pl.ds(off, c_sz)], pbuf)
                pltpu.sync_copy(r_hbm.at[brow, pl.ds(off, c_sz)], rbuf)
                pvals = []
                for ss in range(s):
                    doff = drafts[ss] - off
                    inch = (doff >= 0) & (doff < c_sz)
                    dc = jnp.clip(doff, 0, c_sz - 1)
                    rowi = jnp.full((_L,), ss, jnp.int32)
                    coli = jnp.full((_L,), 0, jnp.int32) + dc
                    val = plsc.load_gather(pbuf, [rowi, coli])
                    lane0 = iota == 0
                    plsc.store_scatter(pbuf, [rowi, coli], zero16,
                                       mask=lane0 & inch)
                    pval = jnp.where(inch, val, zero16)
                    pvals.append(pval)
                    gathv = jnp.where((iota == ss) & inch, val, gathv)

                def vec_body(j, accs):
                    base = j * _L
                    rv = rbuf[pl.ds(base, _L)]
                    ps = [pbuf[ss, pl.ds(base, _L)] for ss in range(s1)]
                    nxt = [jnp.maximum(accs[ss], ps[ss]) for ss in range(s1)]
                    nxt += [jnp.maximum(accs[s1 + ss], ps[ss] * rv)
                            for ss in range(s)]
                    return tuple(nxt)

                accs = lax.fori_loop(0, nvec, vec_body, (zero16,) * nstr)
                new_gaccs = []
                for k in range(nstr):
                    a = accs[k]
                    if k < s:
                        a = jnp.maximum(a, jnp.where(iota == 0, pvals[k],
                                                     zero16))
                    cmax[pl.ds((k * nch + ci) * _L, _L)] = a
                    new_gaccs.append(jnp.maximum(gaccs[k], a))
                return (gathv,) + tuple(new_gaccs)

            init = (zero16,) + (zero16,) * nstr
            res = lax.fori_loop(0, nch, chunk_body, init)
            gathv = res[0]
            gaccs = res[1:]

            results = []
            for k in range(nstr):
                m = jnp.max(gaccs[k])

                def fc_body(ci, posv, k=k, m=m):
                    a = cmax[pl.ds((k * nch + ci) * _L, _L)]
                    return jnp.minimum(posv, jnp.where(a == m, ci, _BIG))

                cstar = jnp.min(lax.fori_loop(0, nch, fc_body, bigvec))
                cstar = jnp.minimum(jnp.maximum(cstar, 0), nch - 1)
                off = cstar * c_sz
                sp = k if k < s1 else k - s1
                pltpu.sync_copy(p_hbm.at[brow, sp, pl.ds(off, c_sz)],
                                pbuf.at[0])
                if k >= s1:
                    pltpu.sync_copy(r_hbm.at[brow, pl.ds(off, c_sz)], rbuf)
                    dmask = drafts[k - s1] - off

                def rs_body(j, posv, k=k, m=m,
                            dmask=(drafts[k - s1] - off) if k >= s1 else None):
                    base = j * _L
                    val = pbuf[0, pl.ds(base, _L)]
                    idxv = base + iota
                    if k >= s1:
                        val = val * rbuf[pl.ds(base, _L)]
                        val = jnp.where(idxv == dmask, zero16, val)
                    return jnp.minimum(posv,
                                       jnp.where(val == m, idxv, _BIG))

                posv = lax.fori_loop(0, nvec, rs_body, bigvec)
                idx = off + jnp.min(posv)
                results.append(jnp.minimum(idx, v - 1))
            tam = results[:s1]
            rec = results[s1:]

            greedy = miv[s + 1]
            bonus = miv[s]
            acc = jnp.int32(1)
            na = jnp.int32(0)
            for ss in range(s):
                a_g = (drafts[ss] == tam[ss]).astype(jnp.int32)
                a_r = (gathv[ss] >= mfv[ss]).astype(jnp.int32)
                acc = acc * jnp.where(greedy == 1, a_g, a_r)
                na = na + acc
            na_c = jnp.minimum(na, s - 1)
            rg = jnp.int32(0)
            rr_tok = jnp.int32(0)
            for ss in range(s1):
                rg = jnp.where(na == ss, tam[ss], rg)
            for ss in range(s):
                rr_tok = jnp.where(na_c == ss, rec[ss], rr_tok)
            repl = jnp.where(greedy == 1, rg, rr_tok)
            boundary = jnp.where(na == s, bonus, repl)
            dpad = jnp.where(iota < s, miv, 0)
            navec = jnp.full((_L,), 0, jnp.int32) + na
            out_vec = jnp.where(iota < navec, dpad,
                                jnp.where(iota == navec, bigvec * 0 + boundary,
                                          jnp.full((_L,), -1, jnp.int32)))
            outv[...] = out_vec
            pltpu.sync_copy(outv, out_hbm.at[brow])

    return sc_kernel


def kernel(draft_token_ids, target_probs, bonus_token_ids, is_greedy,
           uniform_probs, exp_noise):
    b, s = draft_token_ids.shape
    v = target_probs.shape[-1]
    r = _make_recip_q(b, v)(exp_noise)
    mi = jnp.zeros((b, 16), jnp.int32)
    mi = mi.at[:, :s].set(draft_token_ids.astype(jnp.int32))
    mi = mi.at[:, s].set(bonus_token_ids.astype(jnp.int32))
    mi = mi.at[:, s + 1].set(is_greedy.astype(jnp.int32))
    mf = jnp.zeros((b, 16), jnp.float32)
    mf = mf.at[:, :s].set(uniform_probs.astype(jnp.float32))
    out16 = _make_sc(b, s, v)(target_probs, r, mi, mf)
    return out16[:, :s + 1].astype(draft_token_ids.dtype)
